# baseline (device time: 318800 ns/iter reference)
import jax
import jax.numpy as jnp
from jax import lax
from jax.experimental import pallas as pl
from jax.experimental.pallas import tpu as pltpu

N_DEV = 4


def kernel(A, B):
    m, k = A.shape
    _, n = B.shape
    chunk = m // N_DEV

    def body(a_ref, b_ref, out_ref, comm_ref, send_sems, recv_sems):
        my_pos = lax.axis_index("i")
        left = lax.rem(my_pos + N_DEV - 1, N_DEV)
        right = lax.rem(my_pos + 1, N_DEV)

        barrier_sem = pltpu.get_barrier_semaphore()
        for nbr in (left, right):
            pl.semaphore_signal(
                barrier_sem, inc=1,
                device_id=(nbr,), device_id_type=pl.DeviceIdType.MESH,
            )
        pl.semaphore_wait(barrier_sem, 2)

        for c in range(N_DEV):
            out_ref[pl.ds(c * chunk, chunk), :] = jnp.dot(
                a_ref[pl.ds(c * chunk, chunk), :],
                b_ref[:, :],
                preferred_element_type=jnp.float32,
            )

        for s in range(N_DEV - 1):
            send_c = lax.rem(my_pos + N_DEV - s, N_DEV)
            recv_c = lax.rem(my_pos + N_DEV - s - 1, N_DEV)
            slot = s % 2
            rdma = pltpu.make_async_remote_copy(
                src_ref=out_ref.at[pl.ds(send_c * chunk, chunk), :],
                dst_ref=comm_ref.at[slot],
                send_sem=send_sems.at[slot],
                recv_sem=recv_sems.at[slot],
                device_id=(right,),
                device_id_type=pl.DeviceIdType.MESH,
            )
            rdma.start()
            rdma.wait()
            out_ref[pl.ds(recv_c * chunk, chunk), :] = (
                out_ref[pl.ds(recv_c * chunk, chunk), :] + comm_ref[slot]
            )

        own_c = lax.rem(my_pos + 1, N_DEV)
        out_ref[pl.ds(own_c * chunk, chunk), :] = jnp.maximum(
            out_ref[pl.ds(own_c * chunk, chunk), :], 0.0
        )

        for t in range(N_DEV - 1):
            s = N_DEV - 1 + t
            send_c = lax.rem(my_pos + 1 + N_DEV - t, N_DEV)
            slot = s % 2
            rdma = pltpu.make_async_remote_copy(
                src_ref=out_ref.at[pl.ds(send_c * chunk, chunk), :],
                dst_ref=out_ref.at[pl.ds(send_c * chunk, chunk), :],
                send_sem=send_sems.at[slot],
                recv_sem=recv_sems.at[slot],
                device_id=(right,),
                device_id_type=pl.DeviceIdType.MESH,
            )
            rdma.start()
            rdma.wait()

    return pl.pallas_call(
        body,
        out_shape=jax.ShapeDtypeStruct((m, n), jnp.float32),
        in_specs=[
            pl.BlockSpec(memory_space=pltpu.VMEM),
            pl.BlockSpec(memory_space=pltpu.VMEM),
        ],
        out_specs=pl.BlockSpec(memory_space=pltpu.VMEM),
        scratch_shapes=[
            pltpu.VMEM((2, chunk, n), jnp.float32),
            pltpu.SemaphoreType.DMA((2,)),
            pltpu.SemaphoreType.DMA((2,)),
        ],
        compiler_params=pltpu.CompilerParams(collective_id=0),
    )(A, B)


# device time: 119066 ns/iter; 2.6775x vs baseline; 2.6775x over previous
import jax
import jax.numpy as jnp
from jax import lax
from jax.experimental import pallas as pl
from jax.experimental.pallas import tpu as pltpu

N_DEV = 4


def kernel(A, B):
    m, k = A.shape
    _, n = B.shape
    chunk = m // N_DEV
    half = chunk // 2

    def top(c):
        return pl.ds(c * chunk, half)

    def bot(c):
        return pl.ds(c * chunk + half, half)

    def body(a_ref, b_ref, out_ref,
             send_cw, recv_cw, send_ccw, recv_ccw,
             ss_cw, rs_cw, ss_ccw, rs_ccw):
        my_pos = lax.axis_index("i")
        left = lax.rem(my_pos + N_DEV - 1, N_DEV)
        right = lax.rem(my_pos + 1, N_DEV)

        barrier_sem = pltpu.get_barrier_semaphore()
        for nbr in (left, right):
            pl.semaphore_signal(
                barrier_sem, inc=1,
                device_id=(nbr,), device_id_type=pl.DeviceIdType.MESH,
            )
        pl.semaphore_wait(barrier_sem, 2)

        for c in range(N_DEV):
            out_ref[pl.ds(c * chunk, chunk), :] = jnp.dot(
                a_ref[pl.ds(c * chunk, chunk), :],
                b_ref[:, :],
                preferred_element_type=jnp.float32,
            )

        def hop(s, cw_c, ccw_c):
            slot = s % 2
            send_cw[slot] = out_ref[top(cw_c), :].astype(jnp.bfloat16)
            send_ccw[slot] = out_ref[bot(ccw_c), :].astype(jnp.bfloat16)
            r_cw = pltpu.make_async_remote_copy(
                src_ref=send_cw.at[slot], dst_ref=recv_cw.at[slot],
                send_sem=ss_cw.at[slot], recv_sem=rs_cw.at[slot],
                device_id=(right,), device_id_type=pl.DeviceIdType.MESH,
            )
            r_ccw = pltpu.make_async_remote_copy(
                src_ref=send_ccw.at[slot], dst_ref=recv_ccw.at[slot],
                send_sem=ss_ccw.at[slot], recv_sem=rs_ccw.at[slot],
                device_id=(left,), device_id_type=pl.DeviceIdType.MESH,
            )
            r_cw.start()
            r_ccw.start()
            return r_cw, r_ccw

        for s in range(N_DEV - 1):
            slot = s % 2
            cw_send = lax.rem(my_pos + N_DEV - s, N_DEV)
            cw_recv = lax.rem(my_pos + N_DEV - s - 1, N_DEV)
            ccw_send = lax.rem(my_pos + s, N_DEV)
            ccw_recv = lax.rem(my_pos + s + 1, N_DEV)
            r_cw, r_ccw = hop(s, cw_send, ccw_send)
            r_cw.wait()
            out_ref[top(cw_recv), :] = (
                out_ref[top(cw_recv), :] + recv_cw[slot].astype(jnp.float32)
            )
            r_ccw.wait()
            out_ref[bot(ccw_recv), :] = (
                out_ref[bot(ccw_recv), :] + recv_ccw[slot].astype(jnp.float32)
            )

        own_cw = lax.rem(my_pos + 1, N_DEV)
        own_ccw = lax.rem(my_pos + N_DEV - 1, N_DEV)
        out_ref[top(own_cw), :] = jnp.maximum(out_ref[top(own_cw), :], 0.0)
        out_ref[bot(own_ccw), :] = jnp.maximum(out_ref[bot(own_ccw), :], 0.0)

        for t in range(N_DEV - 1):
            s = N_DEV - 1 + t
            slot = s % 2
            cw_send = lax.rem(my_pos + 1 + N_DEV - t, N_DEV)
            cw_recv = lax.rem(my_pos + N_DEV - t, N_DEV)
            ccw_send = lax.rem(my_pos + N_DEV - 1 + t, N_DEV)
            ccw_recv = lax.rem(my_pos + t, N_DEV)
            r_cw, r_ccw = hop(s, cw_send, ccw_send)
            r_cw.wait()
            out_ref[top(cw_recv), :] = recv_cw[slot].astype(jnp.float32)
            r_ccw.wait()
            out_ref[bot(ccw_recv), :] = recv_ccw[slot].astype(jnp.float32)

    comm = lambda: pltpu.VMEM((2, half, n), jnp.bfloat16)
    return pl.pallas_call(
        body,
        out_shape=jax.ShapeDtypeStruct((m, n), jnp.float32),
        in_specs=[
            pl.BlockSpec(memory_space=pltpu.VMEM),
            pl.BlockSpec(memory_space=pltpu.VMEM),
        ],
        out_specs=pl.BlockSpec(memory_space=pltpu.VMEM),
        scratch_shapes=[
            comm(), comm(), comm(), comm(),
            pltpu.SemaphoreType.DMA((2,)),
            pltpu.SemaphoreType.DMA((2,)),
            pltpu.SemaphoreType.DMA((2,)),
            pltpu.SemaphoreType.DMA((2,)),
        ],
        compiler_params=pltpu.CompilerParams(collective_id=0),
    )(A, B)


# device time: 110351 ns/iter; 2.8890x vs baseline; 1.0790x over previous
import jax
import jax.numpy as jnp
from jax import lax
from jax.experimental import pallas as pl
from jax.experimental.pallas import tpu as pltpu

N_DEV = 4


def kernel(A, B):
    m, k = A.shape
    _, n = B.shape
    chunk = m // N_DEV
    half = chunk // 2

    def top(c):
        return pl.ds(c * chunk, half)

    def bot(c):
        return pl.ds(c * chunk + half, half)

    f32 = jnp.float32
    bf16 = jnp.bfloat16

    def body(a_ref, b_ref, out_ref,
             send_cw, recv_cw, send_ccw, recv_ccw,
             ss_cw, rs_cw, ss_ccw, rs_ccw):
        my_pos = lax.axis_index("i")
        left = lax.rem(my_pos + N_DEV - 1, N_DEV)
        right = lax.rem(my_pos + 1, N_DEV)

        def at(pos):
            return lax.rem(my_pos + N_DEV + pos, N_DEV)

        barrier_sem = pltpu.get_barrier_semaphore()
        for nbr in (left, right):
            pl.semaphore_signal(
                barrier_sem, inc=1,
                device_id=(nbr,), device_id_type=pl.DeviceIdType.MESH,
            )
        pl.semaphore_wait(barrier_sem, 2)

        def compute(c):
            out_ref[pl.ds(c * chunk, chunk), :] = jnp.dot(
                a_ref[pl.ds(c * chunk, chunk), :], b_ref[:, :],
                preferred_element_type=f32,
            )

        def start_pair(send_slot, recv_slot):
            r_cw = pltpu.make_async_remote_copy(
                src_ref=send_cw.at[send_slot] if send_slot is not None
                else recv_cw.at[recv_slot + 1],
                dst_ref=recv_cw.at[recv_slot],
                send_sem=ss_cw.at[recv_slot % 2],
                recv_sem=rs_cw.at[recv_slot],
                device_id=(right,), device_id_type=pl.DeviceIdType.MESH,
            )
            r_ccw = pltpu.make_async_remote_copy(
                src_ref=send_ccw.at[send_slot] if send_slot is not None
                else recv_ccw.at[recv_slot + 1],
                dst_ref=recv_ccw.at[recv_slot],
                send_sem=ss_ccw.at[recv_slot % 2],
                recv_sem=rs_ccw.at[recv_slot],
                device_id=(left,), device_id_type=pl.DeviceIdType.MESH,
            )
            r_cw.start()
            r_ccw.start()
            return r_cw, r_ccw

        compute(at(0))
        send_cw[0] = out_ref[top(at(0)), :].astype(bf16)
        send_ccw[0] = out_ref[bot(at(0)), :].astype(bf16)
        h_cw, h_ccw = start_pair(0, 0)

        compute(at(1))
        compute(at(-1))

        h_cw.wait()
        send_cw[1] = (out_ref[top(at(-1)), :]
                      + recv_cw[0].astype(f32)).astype(bf16)
        h_ccw.wait()
        send_ccw[1] = (out_ref[bot(at(1)), :]
                       + recv_ccw[0].astype(f32)).astype(bf16)
        h_cw, h_ccw = start_pair(1, 1)

        compute(at(2))

        h_cw.wait()
        send_cw[0] = (out_ref[top(at(2)), :]
                      + recv_cw[1].astype(f32)).astype(bf16)
        h_ccw.wait()
        send_ccw[0] = (out_ref[bot(at(2)), :]
                       + recv_ccw[1].astype(f32)).astype(bf16)
        h_cw, h_ccw = start_pair(0, 0)

        h_cw.wait()
        out_ref[top(at(1)), :] = jnp.maximum(
            out_ref[top(at(1)), :] + recv_cw[0].astype(f32), 0.0)
        h_ccw.wait()
        out_ref[bot(at(-1)), :] = jnp.maximum(
            out_ref[bot(at(-1)), :] + recv_ccw[0].astype(f32), 0.0)

        send_cw[1] = out_ref[top(at(1)), :].astype(bf16)
        send_ccw[1] = out_ref[bot(at(-1)), :].astype(bf16)
        h_cw, h_ccw = start_pair(1, 2)
        h_cw.wait()
        h_ccw.wait()

        h_cw, h_ccw = start_pair(None, 1)
        out_ref[top(at(0)), :] = recv_cw[2].astype(f32)
        out_ref[bot(at(0)), :] = recv_ccw[2].astype(f32)
        h_cw.wait()
        h_ccw.wait()

        h_cw, h_ccw = start_pair(None, 0)
        out_ref[top(at(-1)), :] = recv_cw[1].astype(f32)
        out_ref[bot(at(1)), :] = recv_ccw[1].astype(f32)
        h_cw.wait()
        h_ccw.wait()

        out_ref[top(at(2)), :] = recv_cw[0].astype(f32)
        out_ref[bot(at(2)), :] = recv_ccw[0].astype(f32)

    return pl.pallas_call(
        body,
        out_shape=jax.ShapeDtypeStruct((m, n), f32),
        in_specs=[
            pl.BlockSpec(memory_space=pltpu.VMEM),
            pl.BlockSpec(memory_space=pltpu.VMEM),
        ],
        out_specs=pl.BlockSpec(memory_space=pltpu.VMEM),
        scratch_shapes=[
            pltpu.VMEM((2, half, n), bf16),
            pltpu.VMEM((3, half, n), bf16),
            pltpu.VMEM((2, half, n), bf16),
            pltpu.VMEM((3, half, n), bf16),
            pltpu.SemaphoreType.DMA((2,)),
            pltpu.SemaphoreType.DMA((3,)),
            pltpu.SemaphoreType.DMA((2,)),
            pltpu.SemaphoreType.DMA((3,)),
        ],
        compiler_params=pltpu.CompilerParams(collective_id=0),
    )(A, B)
